# Initial kernel scaffold; baseline (speedup 1.0000x reference)
import jax
import jax.numpy as jnp
from jax.experimental import pallas as pl


def kernel(positions, table):
    raise NotImplementedError("write your pallas kernel here")


# speedup vs baseline: 3.3311x; 3.3311x over previous
import functools

import jax
import jax.numpy as jnp
import numpy as np
from jax import lax
from jax.experimental import pallas as pl
from jax.experimental.pallas import tpu as pltpu
from jax.experimental.pallas import tpu_sc as plsc

SPARSE_DIM = 16
GRID_DIM = 8
RES = SPARSE_DIM * GRID_DIM
N_PTS = 1048576

NC = 2
NS = 16
NW = NC * NS
L = 16

C = 1024
PER_W = N_PTS // NW
CHUNKS = PER_W // C

HI = np.float32(RES - 1.0 - 1e-6)
SCALE = np.float32(0.5 * RES)


def _axis_math(p):
    u_raw = (p + 1.0) * SCALE
    u = jnp.minimum(jnp.maximum(u_raw, 0.0), HI)
    b = u.astype(jnp.int32)
    f = u - b.astype(jnp.float32)
    inside = (u_raw > 0.0) & (u_raw < HI)
    edge = (u_raw == 0.0) | (u_raw == HI)
    gf = jnp.where(inside, SCALE, jnp.where(edge, np.float32(0.5) * SCALE, np.float32(0.0)))
    return b, f, gf


def _sc_body(px_hbm, py_hbm, pz_hbm, flat_hbm, emb_hbm, gx_hbm, gy_hbm, gz_hbm,
             px_v, py_v, pz_v, *rest):
    idx_bufs = (rest[0:8], rest[8:16])
    val_bufs = (rest[16:24], rest[24:32])
    f_bufs = (rest[32:35], rest[35:38])
    g_bufs = (rest[38:41], rest[41:44])
    oe_v, ogx_v, ogy_v, ogz_v = rest[44:48]
    sems = rest[48:50]

    wid = lax.axis_index("s") * NC + lax.axis_index("c")

    def stage(t, which):
        idxs, vals, fs, gs, sem = idx_bufs[which], val_bufs[which], f_bufs[which], g_bufs[which], sems[which]
        base = wid * PER_W + t * C
        pltpu.sync_copy(px_hbm.at[pl.ds(base, C)], px_v)
        pltpu.sync_copy(py_hbm.at[pl.ds(base, C)], py_v)
        pltpu.sync_copy(pz_hbm.at[pl.ds(base, C)], pz_v)

        def index_phase(i, carry):
            s = pl.ds(i * L, L)
            bx, fx, gfx = _axis_math(px_v[s])
            by, fy, gfy = _axis_math(py_v[s])
            bz, fz, gfz = _axis_math(pz_v[s])
            x1 = jnp.minimum(bx + 1, RES - 1)
            y1 = jnp.minimum(by + 1, RES - 1)
            z1 = jnp.minimum(bz + 1, RES - 1)
            tx0 = (bx >> 3) << 17 | (bx & 7) << 6
            tx1 = (x1 >> 3) << 17 | (x1 & 7) << 6
            ty0 = (by >> 3) << 13 | (by & 7) << 3
            ty1 = (y1 >> 3) << 13 | (y1 & 7) << 3
            tz0 = (bz >> 3) << 9 | (bz & 7)
            tz1 = (z1 >> 3) << 9 | (z1 & 7)
            idxs[0][s] = tx0 | ty0 | tz0
            idxs[1][s] = tx0 | ty0 | tz1
            idxs[2][s] = tx0 | ty1 | tz0
            idxs[3][s] = tx0 | ty1 | tz1
            idxs[4][s] = tx1 | ty0 | tz0
            idxs[5][s] = tx1 | ty0 | tz1
            idxs[6][s] = tx1 | ty1 | tz0
            idxs[7][s] = tx1 | ty1 | tz1
            fs[0][s] = fx
            fs[1][s] = fy
            fs[2][s] = fz
            gs[0][s] = gfx
            gs[1][s] = gfy
            gs[2][s] = gfz
            return carry

        lax.fori_loop(0, C // L, index_phase, 0)
        for cc in range(8):
            pltpu.async_copy(flat_hbm.at[idxs[cc]], vals[cc], sem)

    def finish(t, which):
        idxs, vals, fs, gs, sem = idx_bufs[which], val_bufs[which], f_bufs[which], g_bufs[which], sems[which]
        for cc in range(8):
            pltpu.make_async_copy(flat_hbm.at[idxs[cc]], vals[cc], sem).wait()

        def value_phase(i, carry):
            s = pl.ds(i * L, L)
            fx, fy, fz = fs[0][s], fs[1][s], fs[2][s]
            gfx, gfy, gfz = gs[0][s], gs[1][s], gs[2][s]
            v = [vals[cc][s] for cc in range(8)]
            wz0, wz1 = 1.0 - fz, fz
            t00 = wz0 * v[0] + wz1 * v[1]
            t01 = wz0 * v[2] + wz1 * v[3]
            t10 = wz0 * v[4] + wz1 * v[5]
            t11 = wz0 * v[6] + wz1 * v[7]
            d00 = v[1] - v[0]
            d01 = v[3] - v[2]
            d10 = v[5] - v[4]
            d11 = v[7] - v[6]
            wy0, wy1 = 1.0 - fy, fy
            r0 = wy0 * t00 + wy1 * t01
            r1 = wy0 * t10 + wy1 * t11
            rz0 = wy0 * d00 + wy1 * d01
            rz1 = wy0 * d10 + wy1 * d11
            ry0 = t01 - t00
            ry1 = t11 - t10
            wx0, wx1 = 1.0 - fx, fx
            oe_v[s] = wx0 * r0 + wx1 * r1
            ogz_v[s] = gfz * (wx0 * rz0 + wx1 * rz1)
            ogy_v[s] = gfy * (wx0 * ry0 + wx1 * ry1)
            ogx_v[s] = gfx * (r1 - r0)
            return carry

        lax.fori_loop(0, C // L, value_phase, 0)
        base = wid * PER_W + t * C
        pltpu.sync_copy(oe_v, emb_hbm.at[pl.ds(base, C)])
        pltpu.sync_copy(ogx_v, gx_hbm.at[pl.ds(base, C)])
        pltpu.sync_copy(ogy_v, gy_hbm.at[pl.ds(base, C)])
        pltpu.sync_copy(ogz_v, gz_hbm.at[pl.ds(base, C)])

    stage(0, 0)

    def body(j, carry):
        t0 = 2 * j
        stage(t0 + 1, 1)
        finish(t0, 0)

        @pl.when(t0 + 2 < CHUNKS)
        def _():
            stage(t0 + 2, 0)

        finish(t0 + 1, 1)
        return carry

    lax.fori_loop(0, CHUNKS // 2, body, 0)


@jax.jit
def kernel(positions, table):
    pos_t = positions.T
    flat = table.reshape(-1)

    mesh = plsc.VectorSubcoreMesh(core_axis_name="c", subcore_axis_name="s")
    run = functools.partial(
        pl.kernel,
        mesh=mesh,
        out_type=(
            jax.ShapeDtypeStruct((N_PTS,), jnp.float32),
            jax.ShapeDtypeStruct((N_PTS,), jnp.float32),
            jax.ShapeDtypeStruct((N_PTS,), jnp.float32),
            jax.ShapeDtypeStruct((N_PTS,), jnp.float32),
        ),
        scratch_types=(
            [pltpu.VMEM((C,), jnp.float32) for _ in range(3)]
            + [pltpu.VMEM((C,), jnp.int32) for _ in range(16)]
            + [pltpu.VMEM((C,), jnp.float32) for _ in range(16)]
            + [pltpu.VMEM((C,), jnp.float32) for _ in range(12)]
            + [pltpu.VMEM((C,), jnp.float32) for _ in range(4)]
            + [pltpu.SemaphoreType.DMA, pltpu.SemaphoreType.DMA]
        ),
    )(_sc_body)
    emb, gx, gy, gz = run(pos_t[0], pos_t[1], pos_t[2], flat)
    mask = jnp.all(jnp.abs(positions) <= 1.0, axis=-1)
    return emb[:, None], jnp.stack([gx, gy, gz], axis=-1), mask
